# Initial kernel scaffold; baseline (speedup 1.0000x reference)
import jax
import jax.numpy as jnp
from jax import lax
from jax.experimental import pallas as pl
from jax.experimental.pallas import tpu as pltpu
from jax.experimental.pallas import tpu_sc as plsc

_NUM_CORES = 2
_NUM_SUBCORES = 16
_NUM_WORKERS = _NUM_CORES * _NUM_SUBCORES

_B = 16384 * 50
_D = 32
_B_PER_W = _B // _NUM_WORKERS
_CHUNK = 1600
_N_CHUNKS = _B_PER_W // _CHUNK


def _gather_body(table_hbm, idx_hbm, out_hbm, idx_v, rows_v, gsem, wsem):
    wid = lax.axis_index("s") * _NUM_CORES + lax.axis_index("c")
    base = wid * _B_PER_W
    pltpu.sync_copy(idx_hbm.at[pl.ds(base, _B_PER_W)], idx_v)

    for i in range(_N_CHUNKS):
        b = i % 2
        if i == 0:
            pltpu.async_copy(
                table_hbm.at[idx_v.at[pl.ds(0, _CHUNK)]], rows_v.at[0], gsem
            ).start()
        pltpu.make_async_copy(
            table_hbm.at[idx_v.at[pl.ds(i * _CHUNK, _CHUNK)]],
            rows_v.at[b],
            gsem,
        ).wait()
        if i + 1 < _N_CHUNKS:
            if i >= 1:
                pltpu.make_async_copy(
                    rows_v.at[1 - b],
                    out_hbm.at[pl.ds(base + (i - 1) * _CHUNK, _CHUNK)],
                    wsem,
                ).wait()
            pltpu.async_copy(
                table_hbm.at[idx_v.at[pl.ds((i + 1) * _CHUNK, _CHUNK)]],
                rows_v.at[1 - b],
                gsem,
            ).start()
        pltpu.async_copy(
            rows_v.at[b],
            out_hbm.at[pl.ds(base + i * _CHUNK, _CHUNK)],
            wsem,
        ).start()
    for i in (_N_CHUNKS - 2, _N_CHUNKS - 1):
        pltpu.make_async_copy(
            rows_v.at[i % 2],
            out_hbm.at[pl.ds(base + i * _CHUNK, _CHUNK)],
            wsem,
        ).wait()


@jax.jit
def _gather(table, idx):
    mesh = plsc.VectorSubcoreMesh(
        core_axis_name="c", subcore_axis_name="s",
        num_cores=_NUM_CORES, num_subcores=_NUM_SUBCORES,
    )
    return pl.kernel(
        _gather_body,
        out_type=jax.ShapeDtypeStruct((_B, _D), jnp.float32),
        mesh=mesh,
        scratch_types=[
            pltpu.VMEM((_B_PER_W,), jnp.int32),
            pltpu.VMEM((2, _CHUNK, _D), jnp.float32),
            pltpu.SemaphoreType.DMA,
            pltpu.SemaphoreType.DMA,
        ],
    )(table, idx)


def kernel(inputs, entity_table, relation_table):
    idx = inputs.reshape(-1).astype(jnp.int32)
    out = _gather(entity_table, idx)
    return out.reshape(inputs.shape + (entity_table.shape[1],))


# speedup vs baseline: 1.1097x; 1.1097x over previous
import jax
import jax.numpy as jnp
from jax import lax
from jax.experimental import pallas as pl
from jax.experimental.pallas import tpu as pltpu
from jax.experimental.pallas import tpu_sc as plsc

_NUM_CORES = 2
_NUM_SUBCORES = 16
_NUM_WORKERS = _NUM_CORES * _NUM_SUBCORES

_B = 16384 * 50
_D = 32
_B_PER_W = _B // _NUM_WORKERS
_CHUNK = 1600
_N_CHUNKS = _B_PER_W // _CHUNK


def _gather_body(
    table_hbm, idx_hbm, out_hbm, idx_v, rows_v, gsem0, gsem1, wsem0, wsem1
):
    gsems = (gsem0, gsem1)
    wsems = (wsem0, wsem1)
    wid = lax.axis_index("s") * _NUM_CORES + lax.axis_index("c")
    base = wid * _B_PER_W
    pltpu.sync_copy(idx_hbm.at[pl.ds(base, _B_PER_W)], idx_v)

    for i in range(_N_CHUNKS):
        b = i % 2
        if i == 0:
            pltpu.async_copy(
                table_hbm.at[idx_v.at[pl.ds(0, _CHUNK)]], rows_v.at[0],
                gsems[0],
            )
        pltpu.make_async_copy(
            table_hbm.at[idx_v.at[pl.ds(i * _CHUNK, _CHUNK)]],
            rows_v.at[b],
            gsems[b],
        ).wait()
        if i + 1 < _N_CHUNKS:
            if i >= 1:
                pltpu.make_async_copy(
                    rows_v.at[1 - b],
                    out_hbm.at[pl.ds(base + (i - 1) * _CHUNK, _CHUNK)],
                    wsems[1 - b],
                ).wait()
            pltpu.async_copy(
                table_hbm.at[idx_v.at[pl.ds((i + 1) * _CHUNK, _CHUNK)]],
                rows_v.at[1 - b],
                gsems[1 - b],
            )
        pltpu.async_copy(
            rows_v.at[b],
            out_hbm.at[pl.ds(base + i * _CHUNK, _CHUNK)],
            wsems[b],
        )
    for i in (_N_CHUNKS - 2, _N_CHUNKS - 1):
        pltpu.make_async_copy(
            rows_v.at[i % 2],
            out_hbm.at[pl.ds(base + i * _CHUNK, _CHUNK)],
            wsems[i % 2],
        ).wait()


@jax.jit
def _gather(table, idx):
    mesh = plsc.VectorSubcoreMesh(
        core_axis_name="c", subcore_axis_name="s",
        num_cores=_NUM_CORES, num_subcores=_NUM_SUBCORES,
    )
    return pl.kernel(
        _gather_body,
        out_type=jax.ShapeDtypeStruct((_B, _D), jnp.float32),
        mesh=mesh,
        scratch_types=[
            pltpu.VMEM((_B_PER_W,), jnp.int32),
            pltpu.VMEM((2, _CHUNK, _D), jnp.float32),
            pltpu.SemaphoreType.DMA,
            pltpu.SemaphoreType.DMA,
            pltpu.SemaphoreType.DMA,
            pltpu.SemaphoreType.DMA,
        ],
        compiler_params=pltpu.CompilerParams(use_tc_tiling_on_sc=False),
    )(table, idx)


def kernel(inputs, entity_table, relation_table):
    idx = inputs.reshape(-1).astype(jnp.int32)
    out = _gather(entity_table, idx)
    return out.reshape(inputs.shape + (entity_table.shape[1],))


# speedup vs baseline: 1.1129x; 1.0029x over previous
import jax
import jax.numpy as jnp
from jax import lax
from jax.experimental import pallas as pl
from jax.experimental.pallas import tpu as pltpu
from jax.experimental.pallas import tpu_sc as plsc

_NUM_CORES = 2
_NUM_SUBCORES = 16
_NUM_WORKERS = _NUM_CORES * _NUM_SUBCORES

_B = 16384 * 50
_D = 32
_B_PER_W = _B // _NUM_WORKERS
_NBUF = 4
_CHUNK = 800
_N_CHUNKS = _B_PER_W // _CHUNK


def _gather_body(table_hbm, idx_hbm, out_hbm, idx_v, rows_v, *sems):
    gsems = sems[:_NBUF]
    wsems = sems[_NBUF:]
    wid = lax.axis_index("s") * _NUM_CORES + lax.axis_index("c")
    base = wid * _B_PER_W
    pltpu.sync_copy(idx_hbm.at[pl.ds(base, _B_PER_W)], idx_v)

    def start_gather(j):
        pltpu.async_copy(
            table_hbm.at[idx_v.at[pl.ds(j * _CHUNK, _CHUNK)]],
            rows_v.at[j % _NBUF],
            gsems[j % _NBUF],
        )

    def writeback(j):
        return pltpu.make_async_copy(
            rows_v.at[j % _NBUF],
            out_hbm.at[pl.ds(base + j * _CHUNK, _CHUNK)],
            wsems[j % _NBUF],
        )

    for j in range(_NBUF - 1):
        start_gather(j)
    for i in range(_N_CHUNKS):
        b = i % _NBUF
        pltpu.make_async_copy(
            table_hbm.at[idx_v.at[pl.ds(i * _CHUNK, _CHUNK)]],
            rows_v.at[b],
            gsems[b],
        ).wait()
        nxt = i + _NBUF - 1
        if nxt < _N_CHUNKS:
            if i >= 1:
                writeback(i - 1).wait()
            start_gather(nxt)
        pltpu.async_copy(
            rows_v.at[b],
            out_hbm.at[pl.ds(base + i * _CHUNK, _CHUNK)],
            wsems[b],
        )
    for i in range(max(0, _N_CHUNKS - _NBUF), _N_CHUNKS):
        writeback(i).wait()


@jax.jit
def _gather(table, idx):
    mesh = plsc.VectorSubcoreMesh(
        core_axis_name="c", subcore_axis_name="s",
        num_cores=_NUM_CORES, num_subcores=_NUM_SUBCORES,
    )
    return pl.kernel(
        _gather_body,
        out_type=jax.ShapeDtypeStruct((_B, _D), jnp.float32),
        mesh=mesh,
        scratch_types=[
            pltpu.VMEM((_B_PER_W,), jnp.int32),
            pltpu.VMEM((_NBUF, _CHUNK, _D), jnp.float32),
        ] + [pltpu.SemaphoreType.DMA] * (2 * _NBUF),
        compiler_params=pltpu.CompilerParams(use_tc_tiling_on_sc=False),
    )(table, idx)


def kernel(inputs, entity_table, relation_table):
    idx = inputs.reshape(-1).astype(jnp.int32)
    out = _gather(entity_table, idx)
    return out.reshape(inputs.shape + (entity_table.shape[1],))


# speedup vs baseline: 1.3759x; 1.2363x over previous
import jax
import jax.numpy as jnp
from jax import lax
from jax.experimental import pallas as pl
from jax.experimental.pallas import tpu as pltpu
from jax.experimental.pallas import tpu_sc as plsc

_NUM_CORES = 2
_NUM_SUBCORES = 16
_NUM_WORKERS = _NUM_CORES * _NUM_SUBCORES

_B = 16384 * 50
_D = 32
_B_PER_W = _B // _NUM_WORKERS
_NBUF = 4
_CHUNK = 800
_N_CHUNKS = _B_PER_W // _CHUNK


def _gather_body(table_hbm, idx_hbm, out_hbm, idx_v, rows_v, *sems):
    gsems = sems[:_NBUF]
    wsems = sems[_NBUF:]
    wid = lax.axis_index("s") * _NUM_CORES + lax.axis_index("c")
    base = wid * _B_PER_W
    pltpu.sync_copy(idx_hbm.at[pl.ds(base, _B_PER_W)], idx_v)

    def start_gather(j):
        pltpu.async_copy(
            table_hbm.at[idx_v.at[pl.ds(j * _CHUNK, _CHUNK)]],
            rows_v.at[j % _NBUF],
            gsems[j % _NBUF],
        )

    def writeback(j):
        return pltpu.make_async_copy(
            rows_v.at[j % _NBUF],
            out_hbm.at[wid * _N_CHUNKS + j],
            wsems[j % _NBUF],
        )

    for j in range(_NBUF - 1):
        start_gather(j)
    for i in range(_N_CHUNKS):
        b = i % _NBUF
        pltpu.make_async_copy(
            table_hbm.at[idx_v.at[pl.ds(i * _CHUNK, _CHUNK)]],
            rows_v.at[b],
            gsems[b],
        ).wait()
        nxt = i + _NBUF - 1
        if nxt < _N_CHUNKS:
            if i >= 1:
                writeback(i - 1).wait()
            start_gather(nxt)
        pltpu.async_copy(
            rows_v.at[b],
            out_hbm.at[wid * _N_CHUNKS + i],
            wsems[b],
        )
    for i in range(max(0, _N_CHUNKS - _NBUF), _N_CHUNKS):
        writeback(i).wait()


@jax.jit
def _gather(table, idx):
    mesh = plsc.VectorSubcoreMesh(
        core_axis_name="c", subcore_axis_name="s",
        num_cores=_NUM_CORES, num_subcores=_NUM_SUBCORES,
    )
    return pl.kernel(
        _gather_body,
        out_type=jax.ShapeDtypeStruct((_B // _CHUNK, _CHUNK, _D), jnp.float32),
        mesh=mesh,
        scratch_types=[
            pltpu.VMEM((_B_PER_W,), jnp.int32),
            pltpu.VMEM((_NBUF, _CHUNK, _D), jnp.float32),
        ] + [pltpu.SemaphoreType.DMA] * (2 * _NBUF),
        compiler_params=pltpu.CompilerParams(use_tc_tiling_on_sc=False),
    )(table, idx)


def kernel(inputs, entity_table, relation_table):
    idx = inputs.reshape(-1).astype(jnp.int32)
    out = _gather(entity_table, idx)
    return out.reshape(inputs.shape + (entity_table.shape[1],))


# speedup vs baseline: 1.7933x; 1.3034x over previous
import jax
import jax.numpy as jnp
from jax import lax
from jax.experimental import pallas as pl
from jax.experimental.pallas import tpu as pltpu
from jax.experimental.pallas import tpu_sc as plsc

_NUM_CORES = 2
_NUM_SUBCORES = 16
_NUM_WORKERS = _NUM_CORES * _NUM_SUBCORES

_SEQ = 50
_BATCH = 16384
_B = _BATCH * _SEQ
_D = 32
_B_PER_W = _B // _NUM_WORKERS
_NBUF = 2
_CHUNK = 1600
_ENT = _CHUNK // _SEQ
_N_CHUNKS = _B_PER_W // _CHUNK


def _gather_body(table_hbm, idx_hbm, out_hbm, idx_v, rows_v, *sems):
    gsems = sems[:_NBUF]
    wsems = sems[_NBUF:]
    wid = lax.axis_index("s") * _NUM_CORES + lax.axis_index("c")
    base = wid * _B_PER_W
    ent_base = wid * (_B_PER_W // _SEQ)
    pltpu.sync_copy(idx_hbm.at[pl.ds(base, _B_PER_W)], idx_v)

    def start_gather(j):
        pltpu.async_copy(
            table_hbm.at[idx_v.at[pl.ds(j * _CHUNK, _CHUNK)]],
            rows_v.at[j % _NBUF],
            gsems[j % _NBUF],
        )

    def writeback_descs(j):
        b = j % _NBUF
        return [
            pltpu.make_async_copy(
                rows_v.at[b, pl.ds(k * _SEQ, _SEQ)],
                out_hbm.at[ent_base + j * _ENT + k],
                wsems[b],
            )
            for k in range(_ENT)
        ]

    for j in range(_NBUF - 1):
        start_gather(j)
    for i in range(_N_CHUNKS):
        b = i % _NBUF
        pltpu.make_async_copy(
            table_hbm.at[idx_v.at[pl.ds(i * _CHUNK, _CHUNK)]],
            rows_v.at[b],
            gsems[b],
        ).wait()
        nxt = i + _NBUF - 1
        if nxt < _N_CHUNKS:
            if i >= 1:
                for d in writeback_descs(i - 1):
                    d.wait()
            start_gather(nxt)
        for d in writeback_descs(i):
            d.start()
    for i in range(max(0, _N_CHUNKS - _NBUF), _N_CHUNKS):
        for d in writeback_descs(i):
            d.wait()


@jax.jit
def _gather(table, idx):
    mesh = plsc.VectorSubcoreMesh(
        core_axis_name="c", subcore_axis_name="s",
        num_cores=_NUM_CORES, num_subcores=_NUM_SUBCORES,
    )
    return pl.kernel(
        _gather_body,
        out_type=jax.ShapeDtypeStruct((_BATCH, _SEQ, _D), jnp.float32),
        mesh=mesh,
        scratch_types=[
            pltpu.VMEM((_B_PER_W,), jnp.int32),
            pltpu.VMEM((_NBUF, _CHUNK, _D), jnp.float32),
        ] + [pltpu.SemaphoreType.DMA] * (2 * _NBUF),
        compiler_params=pltpu.CompilerParams(use_tc_tiling_on_sc=False),
    )(table, idx)


def kernel(inputs, entity_table, relation_table):
    idx = inputs.reshape(-1).astype(jnp.int32)
    return _gather(entity_table, idx)
